# Initial kernel scaffold; baseline (speedup 1.0000x reference)
import jax
import jax.numpy as jnp
from jax.experimental import pallas as pl


def kernel(numeric_features, Wn, bn, gn, bln, Wf, bf, gf, blf, gr, blr, Wr1, br1, Wr2, br2, We1, be1, We2, be2, Wt1, bt1, Wt2, bt2, Wi1, bi1, Wi2, bi2, Wc1, bc1, Wc2, bc2):
    raise NotImplementedError("write your pallas kernel here")


# speedup vs baseline: 2.7894x; 2.7894x over previous
import functools

import jax
import jax.numpy as jnp
from jax.experimental import pallas as pl
from jax.experimental.pallas import tpu as pltpu

B = 4096
D_NUM = 256
NUM_HID = 256
TEXT_DIM = 768
FUSION = 1024
E = 8
N_TRAITS = 5
N_CHAR = 10

M_TILE = 1024
M_TILES = B // M_TILE


def _ln(x, g, b):
    mu = jnp.mean(x, axis=-1, keepdims=True)
    var = jnp.var(x, axis=-1, keepdims=True)
    return (x - mu) / jnp.sqrt(var + 1e-5) * g + b


def _moe_kernel(
    x_ref, Wn_ref, bn_ref, gn_ref, bln_ref, Wfn_ref, bf_ref, gf_ref, blf_ref,
    gr_ref, blr_ref, Wr1_ref, br1_ref, Wr2_ref, br2_ref,
    We1_ref, be1_ref, We2_ref, be2_ref,
    Wt1_ref, bt1_ref, Wt2_ref, bt2_ref,
    Wi1_ref, bi1_ref, Wi2_ref, bi2_ref,
    Wc1_ref, bc1_ref, Wc2_ref, bc2_ref,
    trait_ref, irt_ref, char_ref, lb_ref, ent_ref,
    fused_s, w_s, refined_s, usage_s, ent_s,
):
    m = pl.program_id(0)
    e = pl.program_id(1)

    @pl.when(e == 0)
    def _prologue():
        x = x_ref[...]
        num = jax.nn.relu(_ln(jnp.dot(x, Wn_ref[...]) + bn_ref[...],
                              gn_ref[...], bln_ref[...]))
        fused = jax.nn.relu(_ln(jnp.dot(num, Wfn_ref[...]) + bf_ref[...],
                                gf_ref[...], blf_ref[...]))
        fused_s[...] = fused
        h = _ln(fused, gr_ref[...], blr_ref[...])
        h1 = jax.nn.relu(jnp.dot(h, Wr1_ref[...]) + br1_ref[...])
        logits = jnp.dot(h1, Wr2_ref[...]) + br2_ref[...]
        w = jax.nn.softmax(logits, axis=-1)
        w_s[...] = w

        @pl.when(m == 0)
        def _init_scalars():
            usage_s[...] = jnp.zeros_like(usage_s)
            ent_s[0, 0] = 0.0

        usage_s[...] += jnp.sum(w, axis=0, keepdims=True)
        ent_s[0, 0] += jnp.sum(w * jnp.log(w + 1e-12))
        refined_s[...] = jnp.dot(w, be2_ref[...])

    fused = fused_s[...]
    w = w_s[...]
    eh = jax.nn.relu(jnp.dot(fused, We1_ref[0]) + be1_ref[0])
    lane = jax.lax.broadcasted_iota(jnp.int32, (1, E), 1)
    col = jnp.sum(jnp.where(lane == e, w, 0.0), axis=-1, keepdims=True)
    refined_s[...] += jnp.dot(eh * col, We2_ref[0])

    @pl.when(e == E - 1)
    def _epilogue():
        r = refined_s[...]
        t1 = jax.nn.relu(jnp.dot(r, Wt1_ref[...]) + bt1_ref[...])
        trait_ref[...] = jnp.dot(t1, Wt2_ref[...]) + bt2_ref[...]
        i1 = jax.nn.relu(jnp.dot(r, Wi1_ref[...]) + bi1_ref[...])
        irt_ref[...] = jnp.dot(i1, Wi2_ref[...]) + bi2_ref[...]
        c1 = jax.nn.relu(jnp.dot(r, Wc1_ref[...]) + bc1_ref[...])
        char_ref[...] = jnp.dot(c1, Wc2_ref[...]) + bc2_ref[...]

        @pl.when(m == M_TILES - 1)
        def _scalars():
            mu = usage_s[...] / B
            lb = jnp.mean((mu - 1.0 / E) ** 2)
            lb_ref[...] = jnp.full((1, 1), lb, jnp.float32)
            ent_ref[...] = jnp.full((1, 1), -ent_s[0, 0] / B, jnp.float32)


def kernel(numeric_features, Wn, bn, gn, bln, Wf, bf, gf, blf, gr, blr,
           Wr1, br1, Wr2, br2, We1, be1, We2, be2, Wt1, bt1, Wt2, bt2,
           Wi1, bi1, Wi2, bi2, Wc1, bc1, Wc2, bc2):
    Wfn = Wf[TEXT_DIM:, :]
    row = lambda v: v.reshape(1, -1)

    const = lambda *_: (0, 0)
    by_m = lambda m, e: (m, 0)
    by_e3 = lambda m, e: (e, 0, 0)
    by_e2 = lambda m, e: (e, 0)

    grid = (M_TILES, E)
    out = pl.pallas_call(
        _moe_kernel,
        grid=grid,
        in_specs=[
            pl.BlockSpec((M_TILE, D_NUM), by_m),
            pl.BlockSpec((D_NUM, NUM_HID), const),
            pl.BlockSpec((1, NUM_HID), const),
            pl.BlockSpec((1, NUM_HID), const),
            pl.BlockSpec((1, NUM_HID), const),
            pl.BlockSpec((NUM_HID, FUSION), const),
            pl.BlockSpec((1, FUSION), const),
            pl.BlockSpec((1, FUSION), const),
            pl.BlockSpec((1, FUSION), const),
            pl.BlockSpec((1, FUSION), const),
            pl.BlockSpec((1, FUSION), const),
            pl.BlockSpec((FUSION, FUSION // 2), const),
            pl.BlockSpec((1, FUSION // 2), const),
            pl.BlockSpec((FUSION // 2, E), const),
            pl.BlockSpec((1, E), const),
            pl.BlockSpec((1, FUSION, FUSION), by_e3),
            pl.BlockSpec((1, 1, FUSION), by_e3),
            pl.BlockSpec((1, FUSION, FUSION), by_e3),
            pl.BlockSpec((E, FUSION), const),
            pl.BlockSpec((FUSION, 256), const),
            pl.BlockSpec((1, 256), const),
            pl.BlockSpec((256, N_TRAITS), const),
            pl.BlockSpec((1, N_TRAITS), const),
            pl.BlockSpec((FUSION, 256), const),
            pl.BlockSpec((1, 256), const),
            pl.BlockSpec((256, 3), const),
            pl.BlockSpec((1, 3), const),
            pl.BlockSpec((FUSION, 256), const),
            pl.BlockSpec((1, 256), const),
            pl.BlockSpec((256, N_CHAR), const),
            pl.BlockSpec((1, N_CHAR), const),
        ],
        out_specs=[
            pl.BlockSpec((M_TILE, N_TRAITS), by_m),
            pl.BlockSpec((M_TILE, 3), by_m),
            pl.BlockSpec((M_TILE, N_CHAR), by_m),
            pl.BlockSpec((1, 1), const),
            pl.BlockSpec((1, 1), const),
        ],
        out_shape=[
            jax.ShapeDtypeStruct((B, N_TRAITS), jnp.float32),
            jax.ShapeDtypeStruct((B, 3), jnp.float32),
            jax.ShapeDtypeStruct((B, N_CHAR), jnp.float32),
            jax.ShapeDtypeStruct((1, 1), jnp.float32),
            jax.ShapeDtypeStruct((1, 1), jnp.float32),
        ],
        scratch_shapes=[
            pltpu.VMEM((M_TILE, FUSION), jnp.float32),
            pltpu.VMEM((M_TILE, E), jnp.float32),
            pltpu.VMEM((M_TILE, FUSION), jnp.float32),
            pltpu.VMEM((1, E), jnp.float32),
            pltpu.SMEM((1, 1), jnp.float32),
        ],
        compiler_params=pltpu.CompilerParams(
            dimension_semantics=("arbitrary", "arbitrary"),
        ),
    )(
        numeric_features, Wn, row(bn), row(gn), row(bln), Wfn, row(bf),
        row(gf), row(blf), row(gr), row(blr), Wr1, row(br1), Wr2, row(br2),
        We1, be1.reshape(E, 1, FUSION), We2, be2, Wt1, row(bt1), Wt2, row(bt2),
        Wi1, row(bi1), Wi2, row(bi2), Wc1, row(bc1), Wc2, row(bc2),
    )
    trait, irt, char, lb, ent = out
    return trait, irt, char, lb.reshape(()), ent.reshape(())
